# Initial kernel scaffold; baseline (speedup 1.0000x reference)
import jax
import jax.numpy as jnp
from jax.experimental import pallas as pl


def kernel(indices, tables):
    raise NotImplementedError("write your pallas kernel here")


# speedup vs baseline: 1.4061x; 1.4061x over previous
import functools

import jax
import jax.numpy as jnp
from jax import lax
from jax.experimental import pallas as pl
from jax.experimental.pallas import tpu as pltpu
from jax.experimental.pallas import tpu_sc as plsc

_LANES = 16
_NUM_WORKERS = 32


def _make_gather(num_rows, vocab, hist, num_tables, embed_dim):
    rows_per_w = num_rows // _NUM_WORKERS
    g_per_dma = 128
    dmas_per_chunk = 10
    chunk = g_per_dma * dmas_per_chunk
    n_chunks = rows_per_w // chunk
    assert rows_per_w % chunk == 0 and rows_per_w % _LANES == 0
    per_b = num_tables * hist

    mesh = plsc.VectorSubcoreMesh(core_axis_name="c", subcore_axis_name="s")

    @functools.partial(
        pl.kernel,
        mesh=mesh,
        out_type=jax.ShapeDtypeStruct((num_rows, embed_dim), jnp.float32),
        scratch_types=[
            pltpu.VMEM((rows_per_w,), jnp.int32),
            pltpu.VMEM((chunk, embed_dim), jnp.float32),
            pltpu.SemaphoreType.DMA,
        ],
        compiler_params=pltpu.CompilerParams(use_tc_tiling_on_sc=False),
    )
    def gather_kernel(tab_hbm, idx_hbm, out_hbm, idx_v, rows_v, sem):
        wid = lax.axis_index("s") * 2 + lax.axis_index("c")
        base = pl.multiple_of(wid * rows_per_w, 8)

        pltpu.sync_copy(idx_hbm.at[pl.ds(base, rows_per_w)], idx_v)

        iota = lax.iota(jnp.int32, _LANES)
        per_b_v = jnp.full((_LANES,), per_b, jnp.int32)
        hist_v = jnp.full((_LANES,), hist, jnp.int32)

        def add_body(j, _):
            off = pl.multiple_of(j * _LANES, 8)
            pos = j * _LANES + iota
            f = lax.div(lax.rem(pos, per_b_v), hist_v)
            idx_v[pl.ds(off, _LANES)] = idx_v[pl.ds(off, _LANES)] + f * vocab
            return _

        lax.fori_loop(0, rows_per_w // _LANES, add_body, 0)

        def chunk_body(c, _):
            crow = pl.multiple_of(c * chunk, 8)
            copies = []
            for g in range(dmas_per_chunk):
                src_idx = idx_v.at[pl.ds(crow + g * g_per_dma, g_per_dma)]
                dst = rows_v.at[pl.ds(g * g_per_dma, g_per_dma)]
                copies.append(pltpu.async_copy(tab_hbm.at[src_idx], dst, sem))
            for cp in copies:
                cp.wait()
            pltpu.sync_copy(rows_v, out_hbm.at[pl.ds(base + crow, chunk)])
            return _

        lax.fori_loop(0, n_chunks, chunk_body, 0)

    return gather_kernel


def kernel(indices, tables):
    num_tables, batch, hist = indices.shape
    _, vocab, embed_dim = tables.shape
    num_rows = batch * num_tables * hist

    idx_flat = jnp.transpose(indices, (1, 0, 2)).reshape(num_rows)
    tab_flat = tables.reshape(num_tables * vocab, embed_dim)

    gather = _make_gather(num_rows, vocab, hist, num_tables, embed_dim)
    out = gather(tab_flat, idx_flat.astype(jnp.int32))
    return out.reshape(batch, num_tables * hist * embed_dim)
